# baseline (speedup 1.0000x reference)
import functools

import jax
import jax.numpy as jnp
from jax import lax
from jax.experimental import pallas as pl
from jax.experimental.pallas import tpu as pltpu
from jax.experimental.pallas import tpu_sc as plsc

D_ROW = 4096
D_COL = 64
D_ALL = D_ROW * D_COL
BATCH = 128
K = D_ROW * D_COL
NQ = 8
KQ = K // NQ

NUM_CORES = 2
NUM_SUBCORES = 16
NW = NUM_CORES * NUM_SUBCORES
KW = K // NW
KWQ = KQ // NW
CH = 128
NCHQ = KWQ // CH
FCH = 2048
CB = 2048
IB = 8
JB = 512


def _t1_body(x_ref, o_ref):
    o_ref[...] = x_ref[...].T


def _t2_body0(y_ref, gft_ref, o_ref):
    for u in range(IB):
        o_ref[:, u, :] = y_ref[u].T * gft_ref[u][None, :]


def _t2_body1(y_ref, gft_ref, acc_ref, o_ref):
    del acc_ref
    for u in range(IB):
        o_ref[:, u, :] = y_ref[u].T * gft_ref[u][None, :]


def _gft_kernel(filt, fidx, gft_out, idx_v, gft_v, gsem):
    c = lax.axis_index("c")
    s = lax.axis_index("s")
    kw = (s * NUM_CORES + c) * KW
    pltpu.sync_copy(fidx.at[pl.ds(kw, KW)], idx_v)
    fcps = []
    for q in range(KW // FCH):
        sl = pl.ds(q * FCH, FCH)
        fcps.append(pltpu.async_copy(filt.at[idx_v.at[sl]], gft_v.at[sl], gsem))
    for cp in fcps:
        cp.wait()
    pltpu.sync_copy(gft_v, gft_out.at[pl.ds(kw, KW)])


def _make_gather_kernel(qoff):
    def _gather_kernel(xT, ridx, y_out, idx_v, rows0, rows1, gsem, wsem):
        c = lax.axis_index("c")
        s = lax.axis_index("s")
        kw = (s * NUM_CORES + c) * KWQ
        pltpu.sync_copy(ridx.at[pl.ds(qoff + kw, KWQ)], idx_v)

        def idx_sl(n):
            return idx_v.at[pl.ds(n * CH, CH)]

        def y_sl(n):
            return y_out.at[pl.ds(kw + n * CH, CH)]

        pltpu.async_copy(xT.at[idx_sl(0)], rows0, gsem)

        def step(n, buf, other):
            pltpu.make_async_copy(xT.at[idx_sl(n)], buf, gsem).wait()

            @pl.when(n + 1 < NCHQ)
            def _next():
                @pl.when(n >= 1)
                def _drain_prev_write():
                    pltpu.make_async_copy(other, y_sl(n - 1), wsem).wait()
                pltpu.async_copy(xT.at[idx_sl(n + 1)], other, gsem)

            pltpu.async_copy(buf, y_sl(n), wsem)

        def body(m, carry):
            step(m * 2, rows0, rows1)
            step(m * 2 + 1, rows1, rows0)
            return carry
        lax.fori_loop(0, NCHQ // 2, body, 0)

        pltpu.make_async_copy(rows0, y_sl(NCHQ - 2), wsem).wait()
        pltpu.make_async_copy(rows1, y_sl(NCHQ - 1), wsem).wait()
    return _gather_kernel


@jax.jit
def _mapper(x, filt_flat, ridx, fidx):
    mesh = plsc.VectorSubcoreMesh(
        core_axis_name="c", subcore_axis_name="s",
        num_cores=NUM_CORES, num_subcores=NUM_SUBCORES)
    sc_params = pltpu.CompilerParams(use_tc_tiling_on_sc=True)

    gft = functools.partial(
        pl.kernel,
        out_type=jax.ShapeDtypeStruct((K,), jnp.float32),
        mesh=mesh,
        compiler_params=sc_params,
        scratch_types=[
            pltpu.VMEM((KW,), jnp.int32),
            pltpu.VMEM((KW,), jnp.float32),
            pltpu.SemaphoreType.DMA,
        ],
    )(_gft_kernel)(filt_flat, fidx)

    xT = pl.pallas_call(
        _t1_body,
        grid=(D_ALL // CB,),
        in_specs=[pl.BlockSpec((BATCH, CB), lambda i: (0, i))],
        out_specs=pl.BlockSpec((CB, BATCH), lambda i: (i, 0)),
        out_shape=jax.ShapeDtypeStruct((D_ALL, BATCH), jnp.float32),
    )(x)

    ys = []
    for q in range(NQ):
        ys.append(functools.partial(
            pl.kernel,
            out_type=jax.ShapeDtypeStruct((KQ, BATCH), jnp.float32),
            mesh=mesh,
            compiler_params=sc_params,
            scratch_types=[
                pltpu.VMEM((KWQ,), jnp.int32),
                pltpu.VMEM((CH, BATCH), jnp.float32),
                pltpu.VMEM((CH, BATCH), jnp.float32),
                pltpu.SemaphoreType.DMA,
                pltpu.SemaphoreType.DMA,
            ],
        )(_make_gather_kernel(q * KQ))(xT, ridx))

    nqi = KQ // D_ROW
    gft2 = gft.reshape(D_COL, D_ROW)
    out = None
    for q in range(NQ):
        y3 = ys[q].reshape(nqi, D_ROW, BATCH)
        grid = (nqi // IB, D_ROW // JB)
        in_specs = [pl.BlockSpec((IB, JB, BATCH), lambda ib, jb: (ib, jb, 0)),
                    pl.BlockSpec((IB, JB), lambda ib, jb, q=q: (q * (nqi // IB) + ib, jb))]
        out_spec = pl.BlockSpec((BATCH, IB, JB),
                                lambda ib, jb, q=q: (0, q * (nqi // IB) + ib, jb))
        oshape = jax.ShapeDtypeStruct((BATCH, D_COL, D_ROW), jnp.float32)
        if out is None:
            out = pl.pallas_call(
                _t2_body0, grid=grid, in_specs=in_specs,
                out_specs=out_spec, out_shape=oshape,
            )(y3, gft2)
        else:
            out = pl.pallas_call(
                _t2_body1, grid=grid,
                in_specs=in_specs + [pl.BlockSpec(memory_space=pltpu.MemorySpace.HBM)],
                out_specs=out_spec, out_shape=oshape,
                input_output_aliases={2: 0},
            )(y3, gft2, out)
    return out


def kernel(x, filters, ind):
    x = x.reshape(-1, D_ALL)
    ind = ind.astype(jnp.int32)
    ridx = ind.reshape(-1)
    fidx = (ind + jnp.arange(D_COL, dtype=jnp.int32)[:, None] * D_ALL).reshape(-1)
    out3 = _mapper(x, filters.reshape(-1), ridx, fidx)
    return jnp.transpose(out3, (0, 2, 1))


# speedup vs baseline: 1.0382x; 1.0382x over previous
import functools

import jax
import jax.numpy as jnp
from jax import lax
from jax.experimental import pallas as pl
from jax.experimental.pallas import tpu as pltpu
from jax.experimental.pallas import tpu_sc as plsc

D_ROW = 4096
D_COL = 64
D_ALL = D_ROW * D_COL
BATCH = 128
K = D_ROW * D_COL
NQ = 4
KQ = K // NQ

NUM_CORES = 2
NUM_SUBCORES = 16
NW = NUM_CORES * NUM_SUBCORES
KW = K // NW
KWQ = KQ // NW
CH = 256
NCHQ = KWQ // CH
FCH = 2048
CB = 2048
IB = 8
JB = 512


def _t1_body(x_ref, o_ref):
    o_ref[...] = x_ref[...].T


def _t2_body0(y_ref, gft_ref, o_ref):
    for u in range(IB):
        o_ref[:, u, :] = y_ref[u].T * gft_ref[u][None, :]


def _t2_body1(y_ref, gft_ref, acc_ref, o_ref):
    del acc_ref
    for u in range(IB):
        o_ref[:, u, :] = y_ref[u].T * gft_ref[u][None, :]


def _gft_kernel(filt, fidx, gft_out, idx_v, gft_v, gsem):
    c = lax.axis_index("c")
    s = lax.axis_index("s")
    kw = (s * NUM_CORES + c) * KW
    pltpu.sync_copy(fidx.at[pl.ds(kw, KW)], idx_v)
    fcps = []
    for q in range(KW // FCH):
        sl = pl.ds(q * FCH, FCH)
        fcps.append(pltpu.async_copy(filt.at[idx_v.at[sl]], gft_v.at[sl], gsem))
    for cp in fcps:
        cp.wait()
    pltpu.sync_copy(gft_v, gft_out.at[pl.ds(kw, KW)])


def _make_gather_kernel(qoff):
    def _gather_kernel(xT, ridx, y_out, idx_v, rows0, rows1, gsem, wsem):
        c = lax.axis_index("c")
        s = lax.axis_index("s")
        kw = (s * NUM_CORES + c) * KWQ
        pltpu.sync_copy(ridx.at[pl.ds(qoff + kw, KWQ)], idx_v)

        def idx_sl(n):
            return idx_v.at[pl.ds(n * CH, CH)]

        def y_sl(n):
            return y_out.at[pl.ds(kw + n * CH, CH)]

        pltpu.async_copy(xT.at[idx_sl(0)], rows0, gsem)

        def step(n, buf, other):
            pltpu.make_async_copy(xT.at[idx_sl(n)], buf, gsem).wait()

            @pl.when(n + 1 < NCHQ)
            def _next():
                @pl.when(n >= 1)
                def _drain_prev_write():
                    pltpu.make_async_copy(other, y_sl(n - 1), wsem).wait()
                pltpu.async_copy(xT.at[idx_sl(n + 1)], other, gsem)

            pltpu.async_copy(buf, y_sl(n), wsem)

        def body(m, carry):
            step(m * 2, rows0, rows1)
            step(m * 2 + 1, rows1, rows0)
            return carry
        lax.fori_loop(0, NCHQ // 2, body, 0)

        pltpu.make_async_copy(rows0, y_sl(NCHQ - 2), wsem).wait()
        pltpu.make_async_copy(rows1, y_sl(NCHQ - 1), wsem).wait()
    return _gather_kernel


@jax.jit
def _mapper(x, filt_flat, ridx, fidx):
    mesh = plsc.VectorSubcoreMesh(
        core_axis_name="c", subcore_axis_name="s",
        num_cores=NUM_CORES, num_subcores=NUM_SUBCORES)
    sc_params = pltpu.CompilerParams(use_tc_tiling_on_sc=True)

    gft = functools.partial(
        pl.kernel,
        out_type=jax.ShapeDtypeStruct((K,), jnp.float32),
        mesh=mesh,
        compiler_params=sc_params,
        scratch_types=[
            pltpu.VMEM((KW,), jnp.int32),
            pltpu.VMEM((KW,), jnp.float32),
            pltpu.SemaphoreType.DMA,
        ],
    )(_gft_kernel)(filt_flat, fidx)

    xT = pl.pallas_call(
        _t1_body,
        grid=(D_ALL // CB,),
        in_specs=[pl.BlockSpec((BATCH, CB), lambda i: (0, i))],
        out_specs=pl.BlockSpec((CB, BATCH), lambda i: (i, 0)),
        out_shape=jax.ShapeDtypeStruct((D_ALL, BATCH), jnp.float32),
    )(x)

    ys = []
    for q in range(NQ):
        ys.append(functools.partial(
            pl.kernel,
            out_type=jax.ShapeDtypeStruct((KQ, BATCH), jnp.float32),
            mesh=mesh,
            compiler_params=sc_params,
            scratch_types=[
                pltpu.VMEM((KWQ,), jnp.int32),
                pltpu.VMEM((CH, BATCH), jnp.float32),
                pltpu.VMEM((CH, BATCH), jnp.float32),
                pltpu.SemaphoreType.DMA,
                pltpu.SemaphoreType.DMA,
            ],
        )(_make_gather_kernel(q * KQ))(xT, ridx))

    nqi = KQ // D_ROW
    gft2 = gft.reshape(D_COL, D_ROW)
    out = None
    for q in range(NQ):
        y3 = ys[q].reshape(nqi, D_ROW, BATCH)
        grid = (nqi // IB, D_ROW // JB)
        in_specs = [pl.BlockSpec((IB, JB, BATCH), lambda ib, jb: (ib, jb, 0)),
                    pl.BlockSpec((IB, JB), lambda ib, jb, q=q: (q * (nqi // IB) + ib, jb))]
        out_spec = pl.BlockSpec((BATCH, IB, JB),
                                lambda ib, jb, q=q: (0, q * (nqi // IB) + ib, jb))
        oshape = jax.ShapeDtypeStruct((BATCH, D_COL, D_ROW), jnp.float32)
        if out is None:
            out = pl.pallas_call(
                _t2_body0, grid=grid, in_specs=in_specs,
                out_specs=out_spec, out_shape=oshape,
            )(y3, gft2)
        else:
            out = pl.pallas_call(
                _t2_body1, grid=grid,
                in_specs=in_specs + [pl.BlockSpec(memory_space=pltpu.MemorySpace.HBM)],
                out_specs=out_spec, out_shape=oshape,
                input_output_aliases={2: 0},
            )(y3, gft2, out)
    return out


def kernel(x, filters, ind):
    x = x.reshape(-1, D_ALL)
    ind = ind.astype(jnp.int32)
    ridx = ind.reshape(-1)
    fidx = (ind + jnp.arange(D_COL, dtype=jnp.int32)[:, None] * D_ALL).reshape(-1)
    out3 = _mapper(x, filters.reshape(-1), ridx, fidx)
    return jnp.transpose(out3, (0, 2, 1))


# speedup vs baseline: 1.1359x; 1.0941x over previous
import functools

import jax
import jax.numpy as jnp
from jax import lax
from jax.experimental import pallas as pl
from jax.experimental.pallas import tpu as pltpu
from jax.experimental.pallas import tpu_sc as plsc

D_ROW = 4096
D_COL = 64
D_ALL = D_ROW * D_COL
BATCH = 128
K = D_ROW * D_COL
NQ = 4
KQ = K // NQ

NUM_CORES = 2
NUM_SUBCORES = 16
NW = NUM_CORES * NUM_SUBCORES
KW = K // NW
KWQ = KQ // NW
CH = 256
NCHQ = KWQ // CH
FCH = 2048
CB = 4096
IB = 8
JB = 512


def _t1_body(x_ref, o_ref):
    o_ref[...] = x_ref[...].T


def _t2_body0(y_ref, gft_ref, o_ref):
    for u in range(IB):
        o_ref[:, u, :] = y_ref[u].T * gft_ref[u][None, :]


def _t2_body1(y_ref, gft_ref, acc_ref, o_ref):
    del acc_ref
    for u in range(IB):
        o_ref[:, u, :] = y_ref[u].T * gft_ref[u][None, :]


def _gft_kernel(filt, fidx, gft_out, idx_v, gft_v, gsem):
    c = lax.axis_index("c")
    s = lax.axis_index("s")
    kw = (s * NUM_CORES + c) * KW
    pltpu.sync_copy(fidx.at[pl.ds(kw, KW)], idx_v)
    fcps = []
    for q in range(KW // FCH):
        sl = pl.ds(q * FCH, FCH)
        fcps.append(pltpu.async_copy(filt.at[idx_v.at[sl]], gft_v.at[sl], gsem))
    for cp in fcps:
        cp.wait()
    pltpu.sync_copy(gft_v, gft_out.at[pl.ds(kw, KW)])


def _make_gather_kernel(qoff):
    def _gather_kernel(xT, ridx, y_out, idx_v, rows0, rows1, gsem, wsem):
        c = lax.axis_index("c")
        s = lax.axis_index("s")
        kw = (s * NUM_CORES + c) * KWQ
        pltpu.sync_copy(ridx.at[pl.ds(qoff + kw, KWQ)], idx_v)

        def idx_sl(n):
            return idx_v.at[pl.ds(n * CH, CH)]

        def y_sl(n):
            return y_out.at[pl.ds(kw + n * CH, CH)]

        pltpu.async_copy(xT.at[idx_sl(0)], rows0, gsem)

        def step(n, buf, other):
            pltpu.make_async_copy(xT.at[idx_sl(n)], buf, gsem).wait()

            @pl.when(n + 1 < NCHQ)
            def _next():
                @pl.when(n >= 1)
                def _drain_prev_write():
                    pltpu.make_async_copy(other, y_sl(n - 1), wsem).wait()
                pltpu.async_copy(xT.at[idx_sl(n + 1)], other, gsem)

            pltpu.async_copy(buf, y_sl(n), wsem)

        def body(m, carry):
            step(m * 2, rows0, rows1)
            step(m * 2 + 1, rows1, rows0)
            return carry
        lax.fori_loop(0, NCHQ // 2, body, 0)

        pltpu.make_async_copy(rows0, y_sl(NCHQ - 2), wsem).wait()
        pltpu.make_async_copy(rows1, y_sl(NCHQ - 1), wsem).wait()
    return _gather_kernel


@jax.jit
def _mapper(x, filt_flat, ridx, fidx):
    mesh = plsc.VectorSubcoreMesh(
        core_axis_name="c", subcore_axis_name="s",
        num_cores=NUM_CORES, num_subcores=NUM_SUBCORES)
    sc_params = pltpu.CompilerParams(use_tc_tiling_on_sc=True)

    gft = functools.partial(
        pl.kernel,
        out_type=jax.ShapeDtypeStruct((K,), jnp.float32),
        mesh=mesh,
        compiler_params=sc_params,
        scratch_types=[
            pltpu.VMEM((KW,), jnp.int32),
            pltpu.VMEM((KW,), jnp.float32),
            pltpu.SemaphoreType.DMA,
        ],
    )(_gft_kernel)(filt_flat, fidx)

    xT = pl.pallas_call(
        _t1_body,
        grid=(D_ALL // CB,),
        in_specs=[pl.BlockSpec((BATCH, CB), lambda i: (0, i))],
        out_specs=pl.BlockSpec((CB, BATCH), lambda i: (i, 0)),
        out_shape=jax.ShapeDtypeStruct((D_ALL, BATCH), jnp.float32),
    )(x)

    ys = []
    for q in range(NQ):
        ys.append(functools.partial(
            pl.kernel,
            out_type=jax.ShapeDtypeStruct((KQ, BATCH), jnp.float32),
            mesh=mesh,
            compiler_params=sc_params,
            scratch_types=[
                pltpu.VMEM((KWQ,), jnp.int32),
                pltpu.VMEM((CH, BATCH), jnp.float32),
                pltpu.VMEM((CH, BATCH), jnp.float32),
                pltpu.SemaphoreType.DMA,
                pltpu.SemaphoreType.DMA,
            ],
        )(_make_gather_kernel(q * KQ))(xT, ridx))

    nqi = KQ // D_ROW
    gft2 = gft.reshape(D_COL, D_ROW)
    out = None
    for q in range(NQ):
        y3 = ys[q].reshape(nqi, D_ROW, BATCH)
        grid = (nqi // IB, D_ROW // JB)
        in_specs = [pl.BlockSpec((IB, JB, BATCH), lambda ib, jb: (ib, jb, 0)),
                    pl.BlockSpec((IB, JB), lambda ib, jb, q=q: (q * (nqi // IB) + ib, jb))]
        out_spec = pl.BlockSpec((BATCH, IB, JB),
                                lambda ib, jb, q=q: (0, q * (nqi // IB) + ib, jb))
        oshape = jax.ShapeDtypeStruct((BATCH, D_COL, D_ROW), jnp.float32)
        if out is None:
            out = pl.pallas_call(
                _t2_body0, grid=grid, in_specs=in_specs,
                out_specs=out_spec, out_shape=oshape,
            )(y3, gft2)
        else:
            out = pl.pallas_call(
                _t2_body1, grid=grid,
                in_specs=in_specs + [pl.BlockSpec(memory_space=pltpu.MemorySpace.HBM)],
                out_specs=out_spec, out_shape=oshape,
                input_output_aliases={2: 0},
            )(y3, gft2, out)
    return out


def kernel(x, filters, ind):
    x = x.reshape(-1, D_ALL)
    ind = ind.astype(jnp.int32)
    ridx = ind.reshape(-1)
    fidx = (ind + jnp.arange(D_COL, dtype=jnp.int32)[:, None] * D_ALL).reshape(-1)
    out3 = _mapper(x, filters.reshape(-1), ridx, fidx)
    return jnp.transpose(out3, (0, 2, 1))


# speedup vs baseline: 1.1776x; 1.0368x over previous
import functools

import jax
import jax.numpy as jnp
from jax import lax
from jax.experimental import pallas as pl
from jax.experimental.pallas import tpu as pltpu
from jax.experimental.pallas import tpu_sc as plsc

D_ROW = 4096
D_COL = 64
D_ALL = D_ROW * D_COL
BATCH = 128
K = D_ROW * D_COL
NQ = 4
KQ = K // NQ

NUM_CORES = 2
NUM_SUBCORES = 16
NW = NUM_CORES * NUM_SUBCORES
KW = K // NW
KWQ = KQ // NW
CH = 256
NCHQ = KWQ // CH
FCH = 2048
CB = 8192
IB = 8
JB = 512


def _t1_body(x_ref, o_ref):
    o_ref[...] = x_ref[...].T


def _t2_body0(y_ref, gft_ref, o_ref):
    for u in range(IB):
        o_ref[:, u, :] = y_ref[u].T * gft_ref[u][None, :]


def _t2_body1(y_ref, gft_ref, acc_ref, o_ref):
    del acc_ref
    for u in range(IB):
        o_ref[:, u, :] = y_ref[u].T * gft_ref[u][None, :]


def _gft_kernel(filt, fidx, gft_out, idx_v, gft_v, gsem):
    c = lax.axis_index("c")
    s = lax.axis_index("s")
    kw = (s * NUM_CORES + c) * KW
    pltpu.sync_copy(fidx.at[pl.ds(kw, KW)], idx_v)
    fcps = []
    for q in range(KW // FCH):
        sl = pl.ds(q * FCH, FCH)
        fcps.append(pltpu.async_copy(filt.at[idx_v.at[sl]], gft_v.at[sl], gsem))
    for cp in fcps:
        cp.wait()
    pltpu.sync_copy(gft_v, gft_out.at[pl.ds(kw, KW)])


def _make_gather_kernel(qoff):
    def _gather_kernel(xT, ridx, y_out, idx_v, rows0, rows1, gsem, wsem):
        c = lax.axis_index("c")
        s = lax.axis_index("s")
        kw = (s * NUM_CORES + c) * KWQ
        pltpu.sync_copy(ridx.at[pl.ds(qoff + kw, KWQ)], idx_v)

        def idx_sl(n):
            return idx_v.at[pl.ds(n * CH, CH)]

        def y_sl(n):
            return y_out.at[pl.ds(kw + n * CH, CH)]

        pltpu.async_copy(xT.at[idx_sl(0)], rows0, gsem)

        def step(n, buf, other):
            pltpu.make_async_copy(xT.at[idx_sl(n)], buf, gsem).wait()

            @pl.when(n + 1 < NCHQ)
            def _next():
                @pl.when(n >= 1)
                def _drain_prev_write():
                    pltpu.make_async_copy(other, y_sl(n - 1), wsem).wait()
                pltpu.async_copy(xT.at[idx_sl(n + 1)], other, gsem)

            pltpu.async_copy(buf, y_sl(n), wsem)

        def body(m, carry):
            step(m * 2, rows0, rows1)
            step(m * 2 + 1, rows1, rows0)
            return carry
        lax.fori_loop(0, NCHQ // 2, body, 0)

        pltpu.make_async_copy(rows0, y_sl(NCHQ - 2), wsem).wait()
        pltpu.make_async_copy(rows1, y_sl(NCHQ - 1), wsem).wait()
    return _gather_kernel


@jax.jit
def _mapper(x, filt_flat, ridx, fidx):
    mesh = plsc.VectorSubcoreMesh(
        core_axis_name="c", subcore_axis_name="s",
        num_cores=NUM_CORES, num_subcores=NUM_SUBCORES)
    sc_params = pltpu.CompilerParams(use_tc_tiling_on_sc=True)

    gft = functools.partial(
        pl.kernel,
        out_type=jax.ShapeDtypeStruct((K,), jnp.float32),
        mesh=mesh,
        compiler_params=sc_params,
        scratch_types=[
            pltpu.VMEM((KW,), jnp.int32),
            pltpu.VMEM((KW,), jnp.float32),
            pltpu.SemaphoreType.DMA,
        ],
    )(_gft_kernel)(filt_flat, fidx)

    xT = pl.pallas_call(
        _t1_body,
        grid=(D_ALL // CB,),
        in_specs=[pl.BlockSpec((BATCH, CB), lambda i: (0, i))],
        out_specs=pl.BlockSpec((CB, BATCH), lambda i: (i, 0)),
        out_shape=jax.ShapeDtypeStruct((D_ALL, BATCH), jnp.float32),
    )(x)

    ys = []
    for q in range(NQ):
        ys.append(functools.partial(
            pl.kernel,
            out_type=jax.ShapeDtypeStruct((KQ, BATCH), jnp.float32),
            mesh=mesh,
            compiler_params=sc_params,
            scratch_types=[
                pltpu.VMEM((KWQ,), jnp.int32),
                pltpu.VMEM((CH, BATCH), jnp.float32),
                pltpu.VMEM((CH, BATCH), jnp.float32),
                pltpu.SemaphoreType.DMA,
                pltpu.SemaphoreType.DMA,
            ],
        )(_make_gather_kernel(q * KQ))(xT, ridx))

    nqi = KQ // D_ROW
    gft2 = gft.reshape(D_COL, D_ROW)
    out = None
    for q in range(NQ):
        y3 = ys[q].reshape(nqi, D_ROW, BATCH)
        grid = (nqi // IB, D_ROW // JB)
        in_specs = [pl.BlockSpec((IB, JB, BATCH), lambda ib, jb: (ib, jb, 0)),
                    pl.BlockSpec((IB, JB), lambda ib, jb, q=q: (q * (nqi // IB) + ib, jb))]
        out_spec = pl.BlockSpec((BATCH, IB, JB),
                                lambda ib, jb, q=q: (0, q * (nqi // IB) + ib, jb))
        oshape = jax.ShapeDtypeStruct((BATCH, D_COL, D_ROW), jnp.float32)
        if out is None:
            out = pl.pallas_call(
                _t2_body0, grid=grid, in_specs=in_specs,
                out_specs=out_spec, out_shape=oshape,
            )(y3, gft2)
        else:
            out = pl.pallas_call(
                _t2_body1, grid=grid,
                in_specs=in_specs + [pl.BlockSpec(memory_space=pltpu.MemorySpace.HBM)],
                out_specs=out_spec, out_shape=oshape,
                input_output_aliases={2: 0},
            )(y3, gft2, out)
    return out


def kernel(x, filters, ind):
    x = x.reshape(-1, D_ALL)
    ind = ind.astype(jnp.int32)
    ridx = ind.reshape(-1)
    fidx = (ind + jnp.arange(D_COL, dtype=jnp.int32)[:, None] * D_ALL).reshape(-1)
    out3 = _mapper(x, filters.reshape(-1), ridx, fidx)
    return jnp.transpose(out3, (0, 2, 1))


# speedup vs baseline: 1.2009x; 1.0198x over previous
import functools

import jax
import jax.numpy as jnp
from jax import lax
from jax.experimental import pallas as pl
from jax.experimental.pallas import tpu as pltpu
from jax.experimental.pallas import tpu_sc as plsc

D_ROW = 4096
D_COL = 64
D_ALL = D_ROW * D_COL
BATCH = 128
K = D_ROW * D_COL
NQ = 4
KQ = K // NQ

NUM_CORES = 2
NUM_SUBCORES = 16
NW = NUM_CORES * NUM_SUBCORES
KW = K // NW
KWQ = KQ // NW
CH = 256
NCHQ = KWQ // CH
FCH = 2048
CB = 16384
IB = 8
JB = 1024


def _t1_body(x_ref, o_ref):
    o_ref[...] = x_ref[...].T


def _t2_body0(y_ref, gft_ref, o_ref):
    for u in range(IB):
        o_ref[:, u, :] = y_ref[u].T * gft_ref[u][None, :]


def _t2_body1(y_ref, gft_ref, acc_ref, o_ref):
    del acc_ref
    for u in range(IB):
        o_ref[:, u, :] = y_ref[u].T * gft_ref[u][None, :]


def _gft_kernel(filt, fidx, gft_out, idx_v, gft_v, gsem):
    c = lax.axis_index("c")
    s = lax.axis_index("s")
    kw = (s * NUM_CORES + c) * KW
    pltpu.sync_copy(fidx.at[pl.ds(kw, KW)], idx_v)
    fcps = []
    for q in range(KW // FCH):
        sl = pl.ds(q * FCH, FCH)
        fcps.append(pltpu.async_copy(filt.at[idx_v.at[sl]], gft_v.at[sl], gsem))
    for cp in fcps:
        cp.wait()
    pltpu.sync_copy(gft_v, gft_out.at[pl.ds(kw, KW)])


def _make_gather_kernel(qoff):
    def _gather_kernel(xT, ridx, y_out, idx_v, rows0, rows1, gsem, wsem):
        c = lax.axis_index("c")
        s = lax.axis_index("s")
        kw = (s * NUM_CORES + c) * KWQ
        pltpu.sync_copy(ridx.at[pl.ds(qoff + kw, KWQ)], idx_v)

        def idx_sl(n):
            return idx_v.at[pl.ds(n * CH, CH)]

        def y_sl(n):
            return y_out.at[pl.ds(kw + n * CH, CH)]

        pltpu.async_copy(xT.at[idx_sl(0)], rows0, gsem)

        def step(n, buf, other):
            pltpu.make_async_copy(xT.at[idx_sl(n)], buf, gsem).wait()

            @pl.when(n + 1 < NCHQ)
            def _next():
                @pl.when(n >= 1)
                def _drain_prev_write():
                    pltpu.make_async_copy(other, y_sl(n - 1), wsem).wait()
                pltpu.async_copy(xT.at[idx_sl(n + 1)], other, gsem)

            pltpu.async_copy(buf, y_sl(n), wsem)

        def body(m, carry):
            step(m * 2, rows0, rows1)
            step(m * 2 + 1, rows1, rows0)
            return carry
        lax.fori_loop(0, NCHQ // 2, body, 0)

        pltpu.make_async_copy(rows0, y_sl(NCHQ - 2), wsem).wait()
        pltpu.make_async_copy(rows1, y_sl(NCHQ - 1), wsem).wait()
    return _gather_kernel


@jax.jit
def _mapper(x, filt_flat, ridx, fidx):
    mesh = plsc.VectorSubcoreMesh(
        core_axis_name="c", subcore_axis_name="s",
        num_cores=NUM_CORES, num_subcores=NUM_SUBCORES)
    sc_params = pltpu.CompilerParams(use_tc_tiling_on_sc=True)

    gft = functools.partial(
        pl.kernel,
        out_type=jax.ShapeDtypeStruct((K,), jnp.float32),
        mesh=mesh,
        compiler_params=sc_params,
        scratch_types=[
            pltpu.VMEM((KW,), jnp.int32),
            pltpu.VMEM((KW,), jnp.float32),
            pltpu.SemaphoreType.DMA,
        ],
    )(_gft_kernel)(filt_flat, fidx)

    xT = pl.pallas_call(
        _t1_body,
        grid=(D_ALL // CB,),
        in_specs=[pl.BlockSpec((BATCH, CB), lambda i: (0, i))],
        out_specs=pl.BlockSpec((CB, BATCH), lambda i: (i, 0)),
        out_shape=jax.ShapeDtypeStruct((D_ALL, BATCH), jnp.float32),
    )(x)

    ys = []
    for q in range(NQ):
        ys.append(functools.partial(
            pl.kernel,
            out_type=jax.ShapeDtypeStruct((KQ, BATCH), jnp.float32),
            mesh=mesh,
            compiler_params=sc_params,
            scratch_types=[
                pltpu.VMEM((KWQ,), jnp.int32),
                pltpu.VMEM((CH, BATCH), jnp.float32),
                pltpu.VMEM((CH, BATCH), jnp.float32),
                pltpu.SemaphoreType.DMA,
                pltpu.SemaphoreType.DMA,
            ],
        )(_make_gather_kernel(q * KQ))(xT, ridx))

    nqi = KQ // D_ROW
    gft2 = gft.reshape(D_COL, D_ROW)
    out = None
    for q in range(NQ):
        y3 = ys[q].reshape(nqi, D_ROW, BATCH)
        grid = (nqi // IB, D_ROW // JB)
        in_specs = [pl.BlockSpec((IB, JB, BATCH), lambda ib, jb: (ib, jb, 0)),
                    pl.BlockSpec((IB, JB), lambda ib, jb, q=q: (q * (nqi // IB) + ib, jb))]
        out_spec = pl.BlockSpec((BATCH, IB, JB),
                                lambda ib, jb, q=q: (0, q * (nqi // IB) + ib, jb))
        oshape = jax.ShapeDtypeStruct((BATCH, D_COL, D_ROW), jnp.float32)
        if out is None:
            out = pl.pallas_call(
                _t2_body0, grid=grid, in_specs=in_specs,
                out_specs=out_spec, out_shape=oshape,
            )(y3, gft2)
        else:
            out = pl.pallas_call(
                _t2_body1, grid=grid,
                in_specs=in_specs + [pl.BlockSpec(memory_space=pltpu.MemorySpace.HBM)],
                out_specs=out_spec, out_shape=oshape,
                input_output_aliases={2: 0},
            )(y3, gft2, out)
    return out


def kernel(x, filters, ind):
    x = x.reshape(-1, D_ALL)
    ind = ind.astype(jnp.int32)
    ridx = ind.reshape(-1)
    fidx = (ind + jnp.arange(D_COL, dtype=jnp.int32)[:, None] * D_ALL).reshape(-1)
    out3 = _mapper(x, filters.reshape(-1), ridx, fidx)
    return jnp.transpose(out3, (0, 2, 1))
